# baseline (device time: 107845 ns/iter reference)
import jax
import jax.numpy as jnp
from jax import lax
from jax.experimental import pallas as pl
from jax.experimental.pallas import tpu as pltpu

R = 8
ORDERS = ((4, 3, 1), (3, 1, 4), (1, 4, 3))


def kernel(A, B):
    M, K = A.shape
    _, N = B.shape
    m_blk = M // R
    n_q = N // 3

    A = A.astype(jnp.bfloat16)
    B = B.astype(jnp.bfloat16)

    def body(a_ref, b_ref, out_ref,
             sbuf1, rbuf1, sbuf2, rbuf2, sbuf3, rbuf3,
             ssem1, rsem1, ssem2, rsem2, ssem3, rsem3):
        me = lax.axis_index("i")

        barrier = pltpu.get_barrier_semaphore()
        for mask in (1, 3, 4):
            pl.semaphore_signal(barrier, inc=1, device_id=(me ^ mask,),
                                device_id_type=pl.DeviceIdType.MESH)
        pl.semaphore_wait(barrier, 3)

        def partial(b, q):
            return jnp.dot(
                a_ref[pl.ds(b * m_blk, m_blk), :],
                b_ref[:, q * n_q:(q + 1) * n_q],
                preferred_element_type=jnp.float32,
            )

        def start(q, j, src, dst, ssem, rsem, mask):
            rdma = pltpu.make_async_remote_copy(
                src_ref=src.at[q, j], dst_ref=dst.at[q, j],
                send_sem=ssem.at[q, j], recv_sem=rsem.at[q, j],
                device_id=(me ^ mask,),
                device_id_type=pl.DeviceIdType.MESH,
            )
            rdma.start()
            return rdma

        f32 = jnp.float32
        bf = jnp.bfloat16
        rd1 = [[None] * 4 for _ in range(3)]
        rd2 = [[None] * 2 for _ in range(3)]
        rd3 = [None] * 3

        for j in range(4):
            for q, (X1, X2, X3) in enumerate(ORDERS):
                d = (X2 ^ X3, X2, X3, 0)[j]
                sbuf1[q, j] = partial(me ^ X1 ^ d, q).astype(bf)
                rd1[q][j] = start(q, j, sbuf1, rbuf1, ssem1, rsem1, X1)

        for j in range(2):
            for q, (X1, X2, X3) in enumerate(ORDERS):
                b = me ^ (X2 ^ X3 if j == 0 else X2)
                p = partial(b, q).astype(bf)
                rd1[q][j].wait()
                sbuf2[q, j] = p + rbuf1[q, j]
                rd2[q][j] = start(q, j, sbuf2, rbuf2, ssem2, rsem2, X2)

        for q, (X1, X2, X3) in enumerate(ORDERS):
            p = partial(me ^ X3, q).astype(bf)
            rd1[q][2].wait()
            rd2[q][0].wait()
            sbuf3[q, 0] = p + rbuf1[q, 2] + rbuf2[q, 0]
            rd3[q] = start(q, 0, sbuf3, rbuf3, ssem3, rsem3, X3)

        for q, (X1, X2, X3) in enumerate(ORDERS):
            p = partial(me, q)
            rd1[q][3].wait()
            rd2[q][1].wait()
            rd3[q].wait()
            out_ref[:, q * n_q:(q + 1) * n_q] = (
                p + rbuf1[q, 3].astype(f32)
                + rbuf2[q, 1].astype(f32)
                + rbuf3[q, 0].astype(f32)
            )

    out_shape = jax.ShapeDtypeStruct((m_blk, N), jnp.float32)
    bf = jnp.bfloat16
    return pl.pallas_call(
        body,
        out_shape=out_shape,
        in_specs=[pl.BlockSpec(memory_space=pltpu.VMEM),
                  pl.BlockSpec(memory_space=pltpu.VMEM)],
        out_specs=pl.BlockSpec(memory_space=pltpu.VMEM),
        scratch_shapes=[
            pltpu.VMEM((3, 4, m_blk, n_q), bf),
            pltpu.VMEM((3, 4, m_blk, n_q), bf),
            pltpu.VMEM((3, 2, m_blk, n_q), bf),
            pltpu.VMEM((3, 2, m_blk, n_q), bf),
            pltpu.VMEM((3, 1, m_blk, n_q), bf),
            pltpu.VMEM((3, 1, m_blk, n_q), bf),
            pltpu.SemaphoreType.DMA((3, 4)),
            pltpu.SemaphoreType.DMA((3, 4)),
            pltpu.SemaphoreType.DMA((3, 2)),
            pltpu.SemaphoreType.DMA((3, 2)),
            pltpu.SemaphoreType.DMA((3, 1)),
            pltpu.SemaphoreType.DMA((3, 1)),
        ],
        compiler_params=pltpu.CompilerParams(
            collective_id=0,
            vmem_limit_bytes=100 * 1024 * 1024,
        ),
    )(A, B)


# device time: 86647 ns/iter; 1.2446x vs baseline; 1.2446x over previous
import jax
import jax.numpy as jnp
from jax import lax
from jax.experimental import pallas as pl
from jax.experimental.pallas import tpu as pltpu

R = 8
ORDERS = ((4, 3, 1), (3, 1, 4), (1, 4, 3))


def kernel(A, B):
    M, K = A.shape
    _, N = B.shape
    m_blk = M // R
    n_q = N // 3

    def body(a_ref, b_ref, out_ref,
             a_bf, b_bf, astage, bstage,
             sbuf1, rbuf1, rbuf2, rbuf3,
             asem, bsem,
             ssem1, rsem1, ssem2, rsem2, ssem3, rsem3):
        me = lax.axis_index("i")

        barrier = pltpu.get_barrier_semaphore()
        for mask in (1, 3, 4):
            pl.semaphore_signal(barrier, inc=1, device_id=(me ^ mask,),
                                device_id_type=pl.DeviceIdType.MESH)
        pl.semaphore_wait(barrier, 3)

        f32 = jnp.float32
        bf = jnp.bfloat16

        def rows(mask):
            return pl.ds((me ^ mask) * m_blk, m_blk)

        def adma(mask, slot):
            cp = pltpu.make_async_copy(
                a_ref.at[rows(mask), :], astage.at[slot], asem.at[slot])
            cp.start()
            return cp

        def acast(mask, slot, cp):
            cp.wait()
            a_bf[rows(mask), :] = astage[slot].astype(bf)

        def bdma(q):
            cp = pltpu.make_async_copy(
                b_ref.at[:, pl.ds(q * n_q, n_q)], bstage, bsem)
            cp.start()
            return cp

        def bcast(q, cp):
            cp.wait()
            b_bf[:, q * n_q:(q + 1) * n_q] = bstage[...].astype(bf)

        def partial(mask, q):
            return jnp.dot(
                a_bf[rows(mask), :],
                b_bf[:, q * n_q:(q + 1) * n_q],
                preferred_element_type=f32,
            )

        def start(src, dst, ssem, rsem, mask):
            rdma = pltpu.make_async_remote_copy(
                src_ref=src, dst_ref=dst, send_sem=ssem, recv_sem=rsem,
                device_id=(me ^ mask,),
                device_id_type=pl.DeviceIdType.MESH,
            )
            rdma.start()
            return rdma

        rd1 = [[None] * 4 for _ in range(3)]
        rd2 = [[None] * 2 for _ in range(3)]
        rd3 = [None] * 3

        def send1(q, j, blk_mask):
            sbuf1[q, j] = partial(blk_mask, q).astype(bf)
            rd1[q][j] = start(sbuf1.at[q, j], rbuf1.at[q, j],
                              ssem1.at[q, j], rsem1.at[q, j],
                              ORDERS[q][0])

        cb = bdma(0)
        ca = adma(6, 0)
        bcast(0, cb)
        cb = bdma(1)
        acast(6, 0, ca)
        ca = adma(7, 1)
        send1(0, 0, 6)
        bcast(1, cb)
        cb = bdma(2)
        send1(1, 0, 6)
        bcast(2, cb)
        send1(2, 0, 6)
        acast(7, 1, ca)
        ca = adma(2, 0)
        send1(0, 1, 7)
        acast(2, 0, ca)
        ca = adma(5, 1)
        send1(1, 1, 2)
        acast(5, 1, ca)
        ca = adma(4, 0)
        send1(2, 1, 5)
        send1(0, 2, 5)
        acast(4, 0, ca)
        ca = adma(3, 1)
        send1(1, 2, 7)
        acast(3, 1, ca)
        ca = adma(1, 0)
        send1(2, 2, 2)
        acast(1, 0, ca)
        ca = adma(0, 1)
        send1(0, 3, 4)
        send1(1, 3, 3)
        send1(2, 3, 1)

        for j in range(2):
            for q, (X1, X2, X3) in enumerate(ORDERS):
                p = partial(X2 ^ X3 if j == 0 else X2, q).astype(bf)
                rd1[q][j].wait()
                sbuf1[q, j] = p + rbuf1[q, j]
                rd2[q][j] = start(sbuf1.at[q, j], rbuf2.at[q, j],
                                  ssem2.at[q, j], rsem2.at[q, j], X2)

        for q, (X1, X2, X3) in enumerate(ORDERS):
            p = partial(X3, q).astype(bf)
            rd1[q][2].wait()
            rd2[q][0].wait()
            sbuf1[q, 2] = p + rbuf1[q, 2] + rbuf2[q, 0]
            rd3[q] = start(sbuf1.at[q, 2], rbuf3.at[q, 0],
                           ssem3.at[q, 0], rsem3.at[q, 0], X3)

        acast(0, 1, ca)
        for q, (X1, X2, X3) in enumerate(ORDERS):
            p = partial(0, q)
            rd1[q][3].wait()
            rd2[q][1].wait()
            rd3[q].wait()
            out_ref[:, q * n_q:(q + 1) * n_q] = (
                p + rbuf1[q, 3].astype(f32)
                + rbuf2[q, 1].astype(f32)
                + rbuf3[q, 0].astype(f32)
            ).astype(bf)

    out_shape = jax.ShapeDtypeStruct((m_blk, N), jnp.bfloat16)
    bf = jnp.bfloat16
    return pl.pallas_call(
        body,
        out_shape=out_shape,
        in_specs=[pl.BlockSpec(memory_space=pl.ANY),
                  pl.BlockSpec(memory_space=pl.ANY)],
        out_specs=pl.BlockSpec(memory_space=pltpu.VMEM),
        scratch_shapes=[
            pltpu.VMEM((M, K), bf),
            pltpu.VMEM((K, N), bf),
            pltpu.VMEM((2, m_blk, K), jnp.float32),
            pltpu.VMEM((K, n_q), jnp.float32),
            pltpu.VMEM((3, 4, m_blk, n_q), bf),
            pltpu.VMEM((3, 4, m_blk, n_q), bf),
            pltpu.VMEM((3, 2, m_blk, n_q), bf),
            pltpu.VMEM((3, 1, m_blk, n_q), bf),
            pltpu.SemaphoreType.DMA((2,)),
            pltpu.SemaphoreType.DMA,
            pltpu.SemaphoreType.DMA((3, 4)),
            pltpu.SemaphoreType.DMA((3, 4)),
            pltpu.SemaphoreType.DMA((3, 2)),
            pltpu.SemaphoreType.DMA((3, 2)),
            pltpu.SemaphoreType.DMA((3, 1)),
            pltpu.SemaphoreType.DMA((3, 1)),
        ],
        compiler_params=pltpu.CompilerParams(
            collective_id=0,
            vmem_limit_bytes=110 * 1024 * 1024,
        ),
    )(A, B)


# device time: 81834 ns/iter; 1.3179x vs baseline; 1.0588x over previous
import jax
import jax.numpy as jnp
from jax import lax
from jax.experimental import pallas as pl
from jax.experimental.pallas import tpu as pltpu

R = 8
ORDERS = ((4, 3, 1), (3, 1, 4), (1, 4, 3))


def kernel(A, B):
    M, K = A.shape
    _, N = B.shape
    m_blk = M // R
    n_q = N // 3

    def body(a_ref, b_ref, out_ref,
             a_bf, b_bf, astage, bstage,
             sbuf1, rbuf1, rbuf2, rbuf3,
             asem, bsem,
             ssem1, rsem1, ssem2, rsem2, ssem3, rsem3):
        me = lax.axis_index("i")

        barrier = pltpu.get_barrier_semaphore()
        for mask in (1, 3, 4):
            pl.semaphore_signal(barrier, inc=1, device_id=(me ^ mask,),
                                device_id_type=pl.DeviceIdType.MESH)
        pl.semaphore_wait(barrier, 3)

        f32 = jnp.float32
        bf = jnp.bfloat16

        def rows(mask):
            return pl.ds((me ^ mask) * m_blk, m_blk)

        def adma(mask, slot):
            cp = pltpu.make_async_copy(
                a_ref.at[rows(mask), :], astage.at[slot], asem.at[slot])
            cp.start()
            return cp

        def acast(mask, slot, cp):
            cp.wait()
            a_bf[rows(mask), :] = astage[slot].astype(bf)

        def bdma(q):
            cp = pltpu.make_async_copy(
                b_ref.at[:, pl.ds(q * n_q, n_q)], bstage, bsem)
            cp.start()
            return cp

        def bcast(q, cp):
            cp.wait()
            b_bf[:, q * n_q:(q + 1) * n_q] = bstage[...].astype(bf)

        def partial(mask, q):
            return jnp.dot(
                a_bf[rows(mask), :],
                b_bf[:, q * n_q:(q + 1) * n_q],
                preferred_element_type=f32,
            )

        def start(src, dst, ssem, rsem, mask):
            rdma = pltpu.make_async_remote_copy(
                src_ref=src, dst_ref=dst, send_sem=ssem, recv_sem=rsem,
                device_id=(me ^ mask,),
                device_id_type=pl.DeviceIdType.MESH,
            )
            rdma.start()
            return rdma

        rd1 = [[None] * 4 for _ in range(3)]
        rd2 = [[None] * 2 for _ in range(3)]
        rd3 = [None] * 3

        def send1(q, j, blk_mask):
            sbuf1[q, j] = partial(blk_mask, q).astype(bf)
            rd1[q][j] = start(sbuf1.at[q, j], rbuf1.at[q, j],
                              ssem1.at[q, j], rsem1.at[q, j],
                              ORDERS[q][0])

        cb = bdma(0)
        ca = adma(6, 0)
        bcast(0, cb)
        cb = bdma(1)
        acast(6, 0, ca)
        ca = adma(7, 1)
        send1(0, 0, 6)
        bcast(1, cb)
        cb = bdma(2)
        send1(1, 0, 6)
        bcast(2, cb)
        send1(2, 0, 6)
        acast(7, 1, ca)
        ca = adma(2, 0)
        send1(0, 1, 7)
        acast(2, 0, ca)
        ca = adma(5, 1)
        send1(1, 1, 2)
        acast(5, 1, ca)
        ca = adma(4, 0)
        send1(2, 1, 5)
        send1(0, 2, 5)
        acast(4, 0, ca)
        ca = adma(3, 1)
        send1(1, 2, 7)
        acast(3, 1, ca)
        ca = adma(1, 0)
        send1(2, 2, 2)
        acast(1, 0, ca)
        ca = adma(0, 1)
        send1(0, 3, 4)
        send1(1, 3, 3)
        send1(2, 3, 1)

        for j in range(2):
            for q, (X1, X2, X3) in enumerate(ORDERS):
                p = partial(X2 ^ X3 if j == 0 else X2, q).astype(bf)
                rd1[q][j].wait()
                sbuf1[q, j] = p + rbuf1[q, j]
                rd2[q][j] = start(sbuf1.at[q, j], rbuf2.at[q, j],
                                  ssem2.at[q, j], rsem2.at[q, j], X2)

        for q, (X1, X2, X3) in enumerate(ORDERS):
            p = partial(X3, q).astype(bf)
            rd1[q][2].wait()
            rd2[q][0].wait()
            sbuf1[q, 2] = p + rbuf1[q, 2] + rbuf2[q, 0]
            rd3[q] = start(sbuf1.at[q, 2], rbuf3.at[q, 0],
                           ssem3.at[q, 0], rsem3.at[q, 0], X3)

        acast(0, 1, ca)
        for q, (X1, X2, X3) in enumerate(ORDERS):
            p = partial(0, q)
            rd1[q][3].wait()
            rd2[q][1].wait()
            sbuf1[q, 3] = (
                p + rbuf1[q, 3].astype(f32) + rbuf2[q, 1].astype(f32)
            ).astype(bf)
        for q in range(3):
            rd3[q].wait()
            out_ref[:, q * n_q:(q + 1) * n_q] = sbuf1[q, 3] + rbuf3[q, 0]

    out_shape = jax.ShapeDtypeStruct((m_blk, N), jnp.bfloat16)
    bf = jnp.bfloat16
    return pl.pallas_call(
        body,
        out_shape=out_shape,
        in_specs=[pl.BlockSpec(memory_space=pl.ANY),
                  pl.BlockSpec(memory_space=pl.ANY)],
        out_specs=pl.BlockSpec(memory_space=pltpu.VMEM),
        scratch_shapes=[
            pltpu.VMEM((M, K), bf),
            pltpu.VMEM((K, N), bf),
            pltpu.VMEM((2, m_blk, K), jnp.float32),
            pltpu.VMEM((K, n_q), jnp.float32),
            pltpu.VMEM((3, 4, m_blk, n_q), bf),
            pltpu.VMEM((3, 4, m_blk, n_q), bf),
            pltpu.VMEM((3, 2, m_blk, n_q), bf),
            pltpu.VMEM((3, 1, m_blk, n_q), bf),
            pltpu.SemaphoreType.DMA((2,)),
            pltpu.SemaphoreType.DMA,
            pltpu.SemaphoreType.DMA((3, 4)),
            pltpu.SemaphoreType.DMA((3, 4)),
            pltpu.SemaphoreType.DMA((3, 2)),
            pltpu.SemaphoreType.DMA((3, 2)),
            pltpu.SemaphoreType.DMA((3, 1)),
            pltpu.SemaphoreType.DMA((3, 1)),
        ],
        compiler_params=pltpu.CompilerParams(
            collective_id=0,
            vmem_limit_bytes=110 * 1024 * 1024,
        ),
    )(A, B)
